# baseline (speedup 1.0000x reference)
import functools

import jax
import jax.numpy as jnp
from jax import lax
from jax.experimental import pallas as pl
from jax.experimental.pallas import tpu as pltpu
from jax.experimental.pallas import tpu_sc as plsc

B = 4096
H = 200
D = 64
V = 1000000
PD = 128
LANES = 16
NC, NS = 2, 16
NW = NC * NS

_mesh = plsc.VectorSubcoreMesh(core_axis_name="c", subcore_axis_name="s")
_cp = pltpu.CompilerParams(needs_layout_passes=False, use_tc_tiling_on_sc=True)


RB = 128
NBF = V // RB
NPAIR = NBF // NW // 2
TAIL = V - NBF * RB


@functools.partial(
    pl.kernel,
    compiler_params=_cp,
    out_type=jax.ShapeDtypeStruct((V, PD), jnp.float32),
    mesh=_mesh,
    scratch_types=[
        pltpu.VMEM((D, RB), jnp.float32),
        pltpu.VMEM((D, RB), jnp.float32),
        pltpu.VMEM((RB, PD + 1), jnp.float32),
        pltpu.VMEM((RB, PD + 1), jnp.float32),
        pltpu.SemaphoreType.DMA,
        pltpu.SemaphoreType.DMA,
        pltpu.SemaphoreType.DMA,
        pltpu.SemaphoreType.DMA,
    ],
)
def _format_table(tt_hbm, tail_hbm, tp_hbm,
                  in0, in1, tr0, tr1, si0, si1, so0, so1):
    wid = lax.axis_index("s") * NC + lax.axis_index("c")

    def fire_in(blk, in_v, sem):
        pltpu.async_copy(tt_hbm.at[:, pl.ds(blk * RB, RB)], in_v, sem)

    def wait_in(in_v, sem):
        pltpu.make_async_copy(tt_hbm.at[:, pl.ds(0, RB)], in_v, sem).wait()

    iota = lax.iota(jnp.int32, LANES)
    r_vecs = [iota + (rb * LANES) for rb in range(RB // LANES)]

    def transpose(in_v, tr_v):
        @pl.loop(0, D)
        def _(d):
            d_ids = jnp.full((LANES,), 0, jnp.int32) + d
            for rb in range(RB // LANES):
                vals = in_v[d, pl.ds(rb * LANES, LANES)]
                plsc.store_scatter(tr_v, [r_vecs[rb], d_ids], vals)

    def fire_out(blk, tr_v, sem):
        pltpu.async_copy(tr_v.at[:, pl.ds(0, PD)],
                         tp_hbm.at[pl.ds(blk * RB, RB), :], sem)

    def wait_out(tr_v, sem):
        pltpu.make_async_copy(tr_v.at[:, pl.ds(0, PD)],
                              tp_hbm.at[pl.ds(0, RB), :], sem).wait()

    fire_in(wid, in0, si0)

    @pl.loop(0, NPAIR)
    def _(t):
        ba = wid + NW * (2 * t)
        bb = ba + NW
        fire_in(bb, in1, si1)
        wait_in(in0, si0)

        @pl.when(t > 0)
        def _():
            wait_out(tr0, so0)

        transpose(in0, tr0)
        fire_out(ba, tr0, so0)

        @pl.when(t + 1 < NPAIR)
        def _():
            fire_in(bb + NW, in0, si0)

        wait_in(in1, si1)

        @pl.when(t > 0)
        def _():
            wait_out(tr1, so1)

        transpose(in1, tr1)
        fire_out(bb, tr1, so1)

    wait_out(tr0, so0)
    wait_out(tr1, so1)

    @pl.when(wid < NBF - NPAIR * 2 * NW)
    def _():
        blk = NPAIR * 2 * NW + wid
        fire_in(blk, in0, si0)
        wait_in(in0, si0)
        transpose(in0, tr0)
        fire_out(blk, tr0, so0)
        wait_out(tr0, so0)

    @pl.when(wid == 4)
    def _():
        pltpu.sync_copy(tail_hbm, tr0.at[pl.ds(0, TAIL), pl.ds(0, PD)])
        pltpu.sync_copy(tr0.at[pl.ds(0, TAIL), pl.ds(0, PD)],
                        tp_hbm.at[pl.ds(NBF * RB, TAIL), :])


C = 256
KJ = C // 128
NCOL = B // C
TPT = H // 2


@functools.partial(
    pl.kernel,
    compiler_params=_cp,
    out_type=jax.ShapeDtypeStruct((H, D, B), jnp.float32),
    mesh=_mesh,
    scratch_types=[
        pltpu.VMEM((KJ, 128), jnp.int32),
        pltpu.VMEM((KJ, 128), jnp.int32),
        pltpu.VMEM((C, PD), jnp.float32),
        pltpu.VMEM((C, PD), jnp.float32),
        pltpu.VMEM((D, C + 1), jnp.float32),
        pltpu.VMEM((D, C + 1), jnp.float32),
        pltpu.SemaphoreType.DMA,
        pltpu.SemaphoreType.DMA,
        pltpu.SemaphoreType.DMA,
        pltpu.SemaphoreType.DMA,
    ],
)
def _gather_out(tp_hbm, idx_hbm, out_hbm,
                idx0, idx1, rows0, rows1, tr0, tr1,
                sg0, sg1, so0, so1):
    wid = lax.axis_index("s") * NC + lax.axis_index("c")
    b0 = (wid % NCOL) * C
    h_par = wid // NCOL

    def load_and_fire(t, idx_v, rows_v, sem):
        h = h_par + 2 * t
        for j in range(KJ):
            pltpu.sync_copy(idx_hbm.at[h, pl.ds(b0 + j * 128, 128)],
                            idx_v.at[j])
        for j in range(KJ):
            pltpu.async_copy(tp_hbm.at[idx_v.at[j]],
                             rows_v.at[pl.ds(j * 128, 128)], sem)

    def drain_gather(rows_v, sem):
        pltpu.make_async_copy(tp_hbm.at[pl.ds(0, C)], rows_v, sem).wait()

    def fixup(idx_v, rows_v):
        acc = idx_v[0, pl.ds(0, LANES)]
        for g in range(1, C // LANES):
            acc = jnp.minimum(acc, idx_v[g // 8, pl.ds((g % 8) * LANES, LANES)])

        @pl.when(jnp.min(acc) == 0)
        def _():
            zeros = jnp.zeros((LANES,), jnp.float32)
            for g in range(C // LANES):
                vec = idx_v[g // 8, pl.ds((g % 8) * LANES, LANES)]

                @pl.when(jnp.min(vec) == 0)
                def _():
                    mask = vec == 0
                    row_ids = lax.iota(jnp.int32, LANES) + (g * LANES)

                    @pl.loop(0, D)
                    def _(col):
                        col_ids = jnp.full((LANES,), 0, jnp.int32) + col
                        plsc.store_scatter(rows_v, [row_ids, col_ids],
                                           zeros, mask=mask)

    iota = lax.iota(jnp.int32, LANES)
    d_vecs = [iota + (db * LANES) for db in range(D // LANES)]

    def transpose(rows_v, tr_v):
        @pl.loop(0, C)
        def _(b):
            b_ids = jnp.full((LANES,), 0, jnp.int32) + b
            for db in range(D // LANES):
                vals = rows_v[b, pl.ds(db * LANES, LANES)]
                plsc.store_scatter(tr_v, [d_vecs[db], b_ids], vals)

    def fire_out(t, tr_v, sem):
        h = h_par + 2 * t
        pltpu.async_copy(tr_v.at[:, pl.ds(0, C)],
                         out_hbm.at[h, :, pl.ds(b0, C)], sem)

    def wait_out(tr_v, sem):
        pltpu.make_async_copy(tr_v.at[:, pl.ds(0, C)],
                              out_hbm.at[0, :, pl.ds(b0, C)], sem).wait()

    load_and_fire(0, idx0, rows0, sg0)

    @pl.loop(0, TPT // 2)
    def _(tt):
        ta = 2 * tt
        tb = ta + 1
        load_and_fire(tb, idx1, rows1, sg1)
        drain_gather(rows0, sg0)
        fixup(idx0, rows0)

        @pl.when(tt > 0)
        def _():
            wait_out(tr0, so0)

        transpose(rows0, tr0)
        fire_out(ta, tr0, so0)

        @pl.when(tb + 1 < TPT)
        def _():
            load_and_fire(tb + 1, idx0, rows0, sg0)

        drain_gather(rows1, sg1)
        fixup(idx1, rows1)

        @pl.when(tt > 0)
        def _():
            wait_out(tr1, so1)

        transpose(rows1, tr1)
        fire_out(tb, tr1, so1)

    wait_out(tr0, so0)
    wait_out(tr1, so1)


def kernel(x, table):
    tail = jnp.pad(table[V - TAIL:, :], ((0, 0), (0, PD - D)))
    staged = _format_table(table.T, tail)
    idx = x.astype(jnp.int32).T
    out = _gather_out(staged, idx)
    return jnp.transpose(out, (2, 0, 1))


# speedup vs baseline: 3.0979x; 3.0979x over previous
import functools

import jax
import jax.numpy as jnp
from jax import lax
from jax.experimental import pallas as pl
from jax.experimental.pallas import tpu as pltpu
from jax.experimental.pallas import tpu_sc as plsc

B = 4096
H = 200
D = 64
V = 1000000
PD = 128
LANES = 16
NC, NS = 2, 16
NW = NC * NS

_mesh = plsc.VectorSubcoreMesh(core_axis_name="c", subcore_axis_name="s")
_cp = pltpu.CompilerParams(needs_layout_passes=False, use_tc_tiling_on_sc=True)


RB = 128
NBF = V // RB
NPAIR = NBF // NW // 2
TAIL = V - NBF * RB


@functools.partial(
    pl.kernel,
    compiler_params=_cp,
    out_type=jax.ShapeDtypeStruct((V, PD), jnp.float32),
    mesh=_mesh,
    scratch_types=[
        pltpu.VMEM((D, RB), jnp.float32),
        pltpu.VMEM((D, RB), jnp.float32),
        pltpu.VMEM((RB, PD), jnp.float32),
        pltpu.VMEM((RB, PD), jnp.float32),
        pltpu.SemaphoreType.DMA,
        pltpu.SemaphoreType.DMA,
        pltpu.SemaphoreType.DMA,
        pltpu.SemaphoreType.DMA,
    ],
)
def _format_table(tt_hbm, tail_hbm, tp_hbm,
                  in0, in1, tr0, tr1, si0, si1, so0, so1):
    wid = lax.axis_index("s") * NC + lax.axis_index("c")

    def fire_in(blk, in_v, sem):
        pltpu.async_copy(tt_hbm.at[:, pl.ds(blk * RB, RB)], in_v, sem)

    def wait_in(in_v, sem):
        pltpu.make_async_copy(tt_hbm.at[:, pl.ds(0, RB)], in_v, sem).wait()

    iota = lax.iota(jnp.int32, LANES)
    rots = [jnp.bitwise_and(iota + k, LANES - 1) for k in range(LANES)]
    d_vecs = [iota + (db * LANES) for db in range(D // LANES)]

    def transpose(in_v, tr_v):
        @pl.loop(0, RB // LANES)
        def _(rb):
            r0 = rb * LANES
            for k in range(0, LANES, 2):
                r_vec0 = rots[k] + r0
                r_vec1 = rots[k + 1] + r0
                for db in range(D // LANES):
                    vals0 = plsc.load_gather(in_v, [d_vecs[db], r_vec0])
                    vals1 = plsc.load_gather(in_v, [d_vecs[db], r_vec1])
                    plsc.store_scatter(tr_v, [r_vec0, d_vecs[db]], vals0)
                    plsc.store_scatter(tr_v, [r_vec1, d_vecs[db]], vals1)

    def fire_out(blk, tr_v, sem):
        pltpu.async_copy(tr_v, tp_hbm.at[pl.ds(blk * RB, RB), :], sem)

    def wait_out(tr_v, sem):
        pltpu.make_async_copy(tr_v, tp_hbm.at[pl.ds(0, RB), :], sem).wait()

    fire_in(wid, in0, si0)

    @pl.loop(0, NPAIR)
    def _(t):
        ba = wid + NW * (2 * t)
        bb = ba + NW
        fire_in(bb, in1, si1)
        wait_in(in0, si0)

        @pl.when(t > 0)
        def _():
            wait_out(tr0, so0)

        transpose(in0, tr0)
        fire_out(ba, tr0, so0)

        @pl.when(t + 1 < NPAIR)
        def _():
            fire_in(bb + NW, in0, si0)

        wait_in(in1, si1)

        @pl.when(t > 0)
        def _():
            wait_out(tr1, so1)

        transpose(in1, tr1)
        fire_out(bb, tr1, so1)

    wait_out(tr0, so0)
    wait_out(tr1, so1)

    @pl.when(wid < NBF - NPAIR * 2 * NW)
    def _():
        blk = NPAIR * 2 * NW + wid
        fire_in(blk, in0, si0)
        wait_in(in0, si0)
        transpose(in0, tr0)
        fire_out(blk, tr0, so0)
        wait_out(tr0, so0)

    @pl.when(wid == 4)
    def _():
        pltpu.sync_copy(tail_hbm, tr0.at[pl.ds(0, TAIL), :])
        pltpu.sync_copy(tr0.at[pl.ds(0, TAIL), :],
                        tp_hbm.at[pl.ds(NBF * RB, TAIL), :])


C = 256
KJ = C // 128
NCOL = B // C
TPT = H // 2


@functools.partial(
    pl.kernel,
    compiler_params=_cp,
    out_type=jax.ShapeDtypeStruct((H, D, B), jnp.float32),
    mesh=_mesh,
    scratch_types=[
        pltpu.VMEM((KJ, 128), jnp.int32),
        pltpu.VMEM((KJ, 128), jnp.int32),
        pltpu.VMEM((C, PD), jnp.float32),
        pltpu.VMEM((C, PD), jnp.float32),
        pltpu.VMEM((D, C), jnp.float32),
        pltpu.VMEM((D, C), jnp.float32),
        pltpu.SemaphoreType.DMA,
        pltpu.SemaphoreType.DMA,
        pltpu.SemaphoreType.DMA,
        pltpu.SemaphoreType.DMA,
    ],
)
def _gather_out(tp_hbm, idx_hbm, out_hbm,
                idx0, idx1, rows0, rows1, tr0, tr1,
                sg0, sg1, so0, so1):
    wid = lax.axis_index("s") * NC + lax.axis_index("c")
    b0 = (wid % NCOL) * C
    h_par = wid // NCOL

    def load_and_fire(t, idx_v, rows_v, sem):
        h = h_par + 2 * t
        for j in range(KJ):
            pltpu.sync_copy(idx_hbm.at[h, pl.ds(b0 + j * 128, 128)],
                            idx_v.at[j])
        for j in range(KJ):
            pltpu.async_copy(tp_hbm.at[idx_v.at[j]],
                             rows_v.at[pl.ds(j * 128, 128)], sem)

    def drain_gather(rows_v, sem):
        pltpu.make_async_copy(tp_hbm.at[pl.ds(0, C)], rows_v, sem).wait()

    def fixup(idx_v, rows_v):
        acc = idx_v[0, pl.ds(0, LANES)]
        for g in range(1, C // LANES):
            acc = jnp.minimum(acc, idx_v[g // 8, pl.ds((g % 8) * LANES, LANES)])

        @pl.when(jnp.min(acc) == 0)
        def _():
            zeros = jnp.zeros((LANES,), jnp.float32)
            for g in range(C // LANES):
                vec = idx_v[g // 8, pl.ds((g % 8) * LANES, LANES)]

                @pl.when(jnp.min(vec) == 0)
                def _():
                    mask = vec == 0
                    row_ids = lax.iota(jnp.int32, LANES) + (g * LANES)

                    @pl.loop(0, D)
                    def _(col):
                        col_ids = jnp.full((LANES,), 0, jnp.int32) + col
                        plsc.store_scatter(rows_v, [row_ids, col_ids],
                                           zeros, mask=mask)

    iota = lax.iota(jnp.int32, LANES)
    rots = [jnp.bitwise_and(iota + k, LANES - 1) for k in range(LANES)]
    d_vecs = [iota + (db * LANES) for db in range(D // LANES)]

    def transpose(rows_v, tr_v):
        @pl.loop(0, C // LANES)
        def _(bb):
            b0 = bb * LANES
            for k in range(0, LANES, 2):
                b_vec0 = rots[k] + b0
                b_vec1 = rots[k + 1] + b0
                for db in range(D // LANES):
                    vals0 = plsc.load_gather(rows_v, [b_vec0, d_vecs[db]])
                    vals1 = plsc.load_gather(rows_v, [b_vec1, d_vecs[db]])
                    plsc.store_scatter(tr_v, [d_vecs[db], b_vec0], vals0)
                    plsc.store_scatter(tr_v, [d_vecs[db], b_vec1], vals1)

    def fire_out(t, tr_v, sem):
        h = h_par + 2 * t
        pltpu.async_copy(tr_v, out_hbm.at[h, :, pl.ds(b0, C)], sem)

    def wait_out(tr_v, sem):
        pltpu.make_async_copy(tr_v, out_hbm.at[0, :, pl.ds(b0, C)],
                              sem).wait()

    load_and_fire(0, idx0, rows0, sg0)

    @pl.loop(0, TPT // 2)
    def _(tt):
        ta = 2 * tt
        tb = ta + 1
        load_and_fire(tb, idx1, rows1, sg1)
        drain_gather(rows0, sg0)
        fixup(idx0, rows0)

        @pl.when(tt > 0)
        def _():
            wait_out(tr0, so0)

        transpose(rows0, tr0)
        fire_out(ta, tr0, so0)

        @pl.when(tb + 1 < TPT)
        def _():
            load_and_fire(tb + 1, idx0, rows0, sg0)

        drain_gather(rows1, sg1)
        fixup(idx1, rows1)

        @pl.when(tt > 0)
        def _():
            wait_out(tr1, so1)

        transpose(rows1, tr1)
        fire_out(tb, tr1, so1)

    wait_out(tr0, so0)
    wait_out(tr1, so1)


def kernel(x, table):
    tail = jnp.pad(table[V - TAIL:, :], ((0, 0), (0, PD - D)))
    staged = _format_table(table.T, tail)
    idx = x.astype(jnp.int32).T
    out = _gather_out(staged, idx)
    return jnp.transpose(out, (2, 0, 1))
